# baseline (speedup 1.0000x reference)
import functools

import jax
import jax.numpy as jnp
from jax import lax
from jax.experimental import pallas as pl
from jax.experimental.pallas import tpu as pltpu
from jax.experimental.pallas import tpu_sc as plsc

H, W, P, D, EMB = 512, 512, 16, 128, 768
NR = H // P
NC = W // P
N_PATCH = NR * NC
LANES = 16


def _pos_table_sc(row_emb, col_emb):
    mesh = plsc.VectorSubcoreMesh(core_axis_name="c", subcore_axis_name="s")

    @functools.partial(
        pl.kernel,
        mesh=mesh,
        out_type=jax.ShapeDtypeStruct((N_PATCH, EMB), jnp.float32),
        scratch_types=[
            pltpu.VMEM((NC,), jnp.int32),
            pltpu.VMEM((NC, EMB), jnp.float32),
            pltpu.VMEM((1, EMB), jnp.float32),
            pltpu.SemaphoreType.DMA,
        ],
    )
    def k(row_hbm, col_hbm, out_hbm, idx_v, cols_v, row_v, sem):
        tid = lax.axis_index("s") * 2 + lax.axis_index("c")
        for half in range(NC // LANES):
            idx_v[pl.ds(half * LANES, LANES)] = (
                lax.iota(jnp.int32, LANES) + half * LANES
            ) * 4 + 2
        gather = pltpu.async_copy(col_hbm.at[idx_v], cols_v, sem)
        pltpu.sync_copy(row_hbm.at[pl.ds(tid * 4 + 2, 1)], row_v)
        gather.wait()

        def body(c, _):
            for j in range(EMB // LANES):
                sl = pl.ds(j * LANES, LANES)
                cols_v[c, sl] = cols_v[c, sl] + row_v[0, sl]
            return 0

        lax.fori_loop(0, NC, body, 0)
        pltpu.sync_copy(cols_v, out_hbm.at[pl.ds(tid * NC, NC)])

    return k(row_emb, col_emb)


def _add_kernel(x_ref, pos_ref, out_ref):
    out_ref[...] = x_ref[...] + pos_ref[...][None, :, :]


@jax.jit
def kernel(inputs, row_embedding, col_embedding):
    pos = _pos_table_sc(row_embedding, col_embedding)
    B = inputs.shape[0]
    bb = 4
    grid = (B // bb,)
    return pl.pallas_call(
        _add_kernel,
        grid=grid,
        in_specs=[
            pl.BlockSpec((bb, N_PATCH, EMB), lambda i: (i, 0, 0)),
            pl.BlockSpec((N_PATCH, EMB), lambda i: (0, 0)),
        ],
        out_specs=pl.BlockSpec((bb, N_PATCH, EMB), lambda i: (i, 0, 0)),
        out_shape=jax.ShapeDtypeStruct(inputs.shape, inputs.dtype),
    )(inputs, pos)


# speedup vs baseline: 1.1028x; 1.1028x over previous
import functools

import jax
import jax.numpy as jnp
from jax import lax
from jax.experimental import pallas as pl
from jax.experimental.pallas import tpu as pltpu
from jax.experimental.pallas import tpu_sc as plsc

H, W, P, D, EMB = 512, 512, 16, 128, 768
NR = H // P
NC = W // P
N_PATCH = NR * NC
LANES = 16


def _lookup_rows_sc(row_emb, col_emb):
    mesh = plsc.VectorSubcoreMesh(core_axis_name="c", subcore_axis_name="s")

    @functools.partial(
        pl.kernel,
        mesh=mesh,
        out_type=(
            jax.ShapeDtypeStruct((NR, EMB), jnp.float32),
            jax.ShapeDtypeStruct((NC, EMB), jnp.float32),
        ),
        scratch_types=[
            pltpu.VMEM((1, EMB), jnp.float32),
            pltpu.VMEM((1, EMB), jnp.float32),
        ],
    )
    def k(row_hbm, col_hbm, rows_out, cols_out, r_v, c_v):
        tid = lax.axis_index("s") * 2 + lax.axis_index("c")
        src = pl.ds(tid * 4 + 2, 1)
        dst = pl.ds(tid, 1)
        pltpu.sync_copy(row_hbm.at[src], r_v)
        pltpu.sync_copy(col_hbm.at[src], c_v)
        pltpu.sync_copy(r_v, rows_out.at[dst])
        pltpu.sync_copy(c_v, cols_out.at[dst])

    return k(row_emb, col_emb)


def _add_kernel(x_ref, r_ref, c_ref, out_ref, pos_ref):
    @pl.when(pl.program_id(0) == 0)
    def _build():
        pos = r_ref[...][:, None, :] + c_ref[...][None, :, :]
        pos_ref[...] = pos.reshape(N_PATCH, EMB)

    out_ref[...] = x_ref[...] + pos_ref[...][None, :, :]


@jax.jit
def kernel(inputs, row_embedding, col_embedding):
    rows, cols = _lookup_rows_sc(row_embedding, col_embedding)
    B = inputs.shape[0]
    bb = 4
    grid = (B // bb,)
    return pl.pallas_call(
        _add_kernel,
        grid=grid,
        in_specs=[
            pl.BlockSpec((bb, N_PATCH, EMB), lambda i: (i, 0, 0)),
            pl.BlockSpec((NR, EMB), lambda i: (0, 0)),
            pl.BlockSpec((NC, EMB), lambda i: (0, 0)),
        ],
        out_specs=pl.BlockSpec((bb, N_PATCH, EMB), lambda i: (i, 0, 0)),
        out_shape=jax.ShapeDtypeStruct(inputs.shape, inputs.dtype),
        scratch_shapes=[pltpu.VMEM((N_PATCH, EMB), jnp.float32)],
    )(inputs, rows, cols)
